# Initial kernel scaffold; baseline (speedup 1.0000x reference)
import jax
import jax.numpy as jnp
from jax.experimental import pallas as pl
from jax.experimental.pallas import tpu as pltpu

B = 8
K = 2
BK = B * K
HID = 256
SRC = 64
VOCAB = 100000
SOS_ID = 1
MAXLEN = 55
STEPS = MAXLEN

CHUNK = 2048
NCHUNK = (VOCAB + CHUNK - 1) // CHUNK
NEG = jnp.float32(-3.0e38)


def _body(encT_ref, enc2_ref, wx_ref, wh_ref, b_ref, wc_ref, wo_ref, emb_hbm,
          out_ref,
          x_ref, h_ref, c_ref, comb_ref, cum_ref,
          m_ref, s_ref, v1_ref, v2_ref, i1_ref, i2_ref,
          hist_ref, tokv_ref, tok_smem, gsem, csem):
    step = pl.program_id(0)
    ch = pl.program_id(1)

    @pl.when(jnp.logical_and(step == 0, ch == 0))
    def _init():
        h_ref[...] = jnp.zeros((BK, HID), jnp.float32)
        c_ref[...] = jnp.zeros((BK, HID), jnp.float32)
        for b in range(B):
            for j in range(K):
                tok_smem[b, j] = SOS_ID

    @pl.when(ch == 0)
    def _pre():
        copies = [
            pltpu.make_async_copy(
                emb_hbm.at[pl.ds(tok_smem[i // K, i % K], 1), :],
                x_ref.at[pl.ds(i, 1), :],
                gsem,
            )
            for i in range(BK)
        ]
        for cp in copies:
            cp.start()
        for cp in copies:
            cp.wait()

        x = x_ref[...]
        h = h_ref[...]
        c = c_ref[...]
        gates = (jnp.dot(x, wx_ref[...], preferred_element_type=jnp.float32)
                 + jnp.dot(h, wh_ref[...], preferred_element_type=jnp.float32)
                 + b_ref[...])
        i_g = gates[:, 0 * HID:1 * HID]
        f_g = gates[:, 1 * HID:2 * HID]
        g_g = gates[:, 2 * HID:3 * HID]
        o_g = gates[:, 3 * HID:4 * HID]
        c = jax.nn.sigmoid(f_g) * c + jax.nn.sigmoid(i_g) * jnp.tanh(g_g)
        h = jax.nn.sigmoid(o_g) * jnp.tanh(c)
        h_ref[...] = h
        c_ref[...] = c

        full = jnp.dot(h, encT_ref[...], preferred_element_type=jnp.float32)
        row_src = jax.lax.broadcasted_iota(jnp.int32, (BK, B * SRC), 0) // K
        col_src = jax.lax.broadcasted_iota(jnp.int32, (BK, B * SRC), 1) // SRC
        sc = jnp.where(row_src == col_src, full, NEG)
        mx = jnp.max(sc, axis=1, keepdims=True)
        e = jnp.exp(sc - mx)
        attn = e / jnp.sum(e, axis=1, keepdims=True)
        ctx = jnp.dot(attn, enc2_ref[...], preferred_element_type=jnp.float32)
        cat = jnp.concatenate([h, ctx], axis=1)
        comb_ref[...] = jnp.tanh(
            jnp.dot(cat, wc_ref[...], preferred_element_type=jnp.float32))

        m_ref[...] = jnp.full((BK, 1), NEG, jnp.float32)
        s_ref[...] = jnp.zeros((BK, 1), jnp.float32)
        v1_ref[...] = jnp.full((BK, 1), NEG, jnp.float32)
        v2_ref[...] = jnp.full((BK, 1), NEG, jnp.float32)
        i1_ref[...] = jnp.zeros((BK, 1), jnp.int32)
        i2_ref[...] = jnp.zeros((BK, 1), jnp.int32)

    xblk = jnp.dot(comb_ref[...], wo_ref[...],
                   preferred_element_type=jnp.float32)
    colg = ch * CHUNK + jax.lax.broadcasted_iota(jnp.int32, (BK, CHUNK), 1)
    xblk = jnp.where(colg < VOCAB, xblk, NEG)

    cm = jnp.max(xblk, axis=1, keepdims=True)
    m_old = m_ref[...]
    m_new = jnp.maximum(m_old, cm)
    s_ref[...] = (s_ref[...] * jnp.exp(m_old - m_new)
                  + jnp.sum(jnp.exp(xblk - m_new), axis=1, keepdims=True))
    m_ref[...] = m_new

    BIGI = jnp.int32(2 ** 30)
    c1i = jnp.min(jnp.where(xblk == cm, colg, BIGI), axis=1, keepdims=True)
    x2 = jnp.where(colg == c1i, NEG, xblk)
    c2v = jnp.max(x2, axis=1, keepdims=True)
    c2i = jnp.min(jnp.where(x2 == c2v, colg, BIGI), axis=1, keepdims=True)
    c1v = cm

    a1 = v1_ref[...]
    a2 = v2_ref[...]
    ai1 = i1_ref[...]
    ai2 = i2_ref[...]
    awin = a1 >= c1v
    sa = a2 >= c1v
    sb = a1 >= c2v
    v1_ref[...] = jnp.where(awin, a1, c1v)
    i1_ref[...] = jnp.where(awin, ai1, c1i)
    v2_ref[...] = jnp.where(awin, jnp.where(sa, a2, c1v),
                            jnp.where(sb, a1, c2v))
    i2_ref[...] = jnp.where(awin, jnp.where(sa, ai2, c1i),
                            jnp.where(sb, ai1, c2i))

    @pl.when(ch == NCHUNK - 1)
    def _merge():
        lse = jnp.log(s_ref[...])
        m_fin = m_ref[...]
        lp1 = (v1_ref[...] - m_fin) - lse
        lp2 = (v2_ref[...] - m_fin) - lse
        t1 = i1_ref[...]
        t2 = i2_ref[...]
        swap = jnp.logical_and(lp1 == lp2, t1 > t2)
        t1n = jnp.where(swap, t2, t1)
        t2n = jnp.where(swap, t1, t2)

        lp1r = lp1.reshape(B, K)
        lp2r = lp2.reshape(B, K)
        t1r = t1n.reshape(B, K)
        t2r = t2n.reshape(B, K)

        @pl.when(step == 0)
        def _s0():
            cum_ref[...] = jnp.concatenate([lp1r[:, 0:1], lp2r[:, 0:1]], axis=1)
            tk = jnp.concatenate([t1r[:, 0:1], t2r[:, 0:1]], axis=1)
            tokv_ref[...] = tk
            pos = jax.lax.broadcasted_iota(jnp.int32, (B, K, 64), 2)
            hist_ref[...] = jnp.where(pos == 0, tk[:, :, None], 0)

        @pl.when(step > 0)
        def _sn():
            cum = cum_ref[...]
            cand_v = jnp.concatenate(
                [lp1r[:, 0:1] + cum[:, 0:1], lp2r[:, 0:1] + cum[:, 0:1],
                 lp1r[:, 1:2] + cum[:, 1:2], lp2r[:, 1:2] + cum[:, 1:2]],
                axis=1)
            cand_t = jnp.concatenate(
                [t1r[:, 0:1], t2r[:, 0:1], t1r[:, 1:2], t2r[:, 1:2]], axis=1)
            i4 = jax.lax.broadcasted_iota(jnp.int32, (B, 4), 1)
            b1 = jnp.max(cand_v, axis=1, keepdims=True)
            s1 = jnp.min(jnp.where(cand_v == b1, i4, 99), axis=1, keepdims=True)
            cv2 = jnp.where(i4 == s1, NEG, cand_v)
            b2 = jnp.max(cv2, axis=1, keepdims=True)
            s2 = jnp.min(jnp.where(cv2 == b2, i4, 99), axis=1, keepdims=True)
            sel = jnp.concatenate([s1, s2], axis=1)
            newcum = jnp.concatenate([b1, b2], axis=1)
            tk0 = jnp.sum(jnp.where(i4 == s1, cand_t, 0), axis=1, keepdims=True)
            tk1 = jnp.sum(jnp.where(i4 == s2, cand_t, 0), axis=1, keepdims=True)
            tk = jnp.concatenate([tk0, tk1], axis=1)
            previd = sel // K

            hp = hist_ref[...]
            h0 = hp[:, 0:1, :]
            h1 = hp[:, 1:2, :]
            ph = jnp.where((previd == 0)[:, :, None], h0, h1)
            pos = jax.lax.broadcasted_iota(jnp.int32, (B, K, 64), 2)
            hist_ref[...] = jnp.where(pos == step, tk[:, :, None], ph)
            cum_ref[...] = newcum
            tokv_ref[...] = tk

        cp = pltpu.make_async_copy(tokv_ref, tok_smem, csem)
        cp.start()
        cp.wait()

        @pl.when(step == STEPS - 1)
        def _out():
            out_ref[...] = hist_ref[:, :, :MAXLEN]


def kernel(input_var, encoder_outputs, k, emb, Wx, Wh, b, Wc, Wo):
    del input_var, k
    enc2 = encoder_outputs.reshape(B * SRC, HID)
    encT = enc2.T
    b2 = b.reshape(1, 4 * HID)

    grid = (STEPS, NCHUNK)
    beams = pl.pallas_call(
        _body,
        grid=grid,
        in_specs=[
            pl.BlockSpec((HID, B * SRC), lambda s, c: (0, 0)),
            pl.BlockSpec((B * SRC, HID), lambda s, c: (0, 0)),
            pl.BlockSpec((HID, 4 * HID), lambda s, c: (0, 0)),
            pl.BlockSpec((HID, 4 * HID), lambda s, c: (0, 0)),
            pl.BlockSpec((1, 4 * HID), lambda s, c: (0, 0)),
            pl.BlockSpec((2 * HID, HID), lambda s, c: (0, 0)),
            pl.BlockSpec((HID, CHUNK), lambda s, c: (0, c)),
            pl.BlockSpec(memory_space=pltpu.ANY),
        ],
        out_specs=pl.BlockSpec((B, K, MAXLEN), lambda s, c: (0, 0, 0)),
        out_shape=jax.ShapeDtypeStruct((B, K, MAXLEN), jnp.int32),
        scratch_shapes=[
            pltpu.VMEM((BK, HID), jnp.float32),
            pltpu.VMEM((BK, HID), jnp.float32),
            pltpu.VMEM((BK, HID), jnp.float32),
            pltpu.VMEM((BK, HID), jnp.float32),
            pltpu.VMEM((B, K), jnp.float32),
            pltpu.VMEM((BK, 1), jnp.float32),
            pltpu.VMEM((BK, 1), jnp.float32),
            pltpu.VMEM((BK, 1), jnp.float32),
            pltpu.VMEM((BK, 1), jnp.float32),
            pltpu.VMEM((BK, 1), jnp.int32),
            pltpu.VMEM((BK, 1), jnp.int32),
            pltpu.VMEM((B, K, 64), jnp.int32),
            pltpu.VMEM((B, K), jnp.int32),
            pltpu.SMEM((B, K), jnp.int32),
            pltpu.SemaphoreType.DMA,
            pltpu.SemaphoreType.DMA,
        ],
        compiler_params=pltpu.CompilerParams(
            dimension_semantics=("arbitrary", "arbitrary"),
        ),
    )(encT, enc2, Wx, Wh, b2, Wc, Wo, emb)
    return beams


# speedup vs baseline: 90.3156x; 90.3156x over previous
import jax
import jax.numpy as jnp
from jax.experimental import pallas as pl
from jax.experimental.pallas import tpu as pltpu

B = 8
K = 2
BK = B * K
HID = 256
SRC = 64
VOCAB = 100000
SOS_ID = 1
MAXLEN = 55
STEPS = MAXLEN

CHUNK = 2048
NCHUNK = (VOCAB + CHUNK - 1) // CHUNK
VSTORE = NCHUNK * CHUNK
VPAD = ((VOCAB + 127) // 128) * 128
NEG = -3.0e38


def _body(encT_ref, enc2_ref, wx_ref, wh_ref, b_ref, wc_ref, wo_ref, emb_hbm,
          out0_ref, out1_ref,
          x_ref, h_ref, c_ref, comb_ref, cum_ref, logits_ref,
          m_ref, v1_ref, v2_ref, i1_ref, i2_ref,
          hist_ref, tokv_ref, tok_smem, gsem, csem):
    step = pl.program_id(0)
    ch = pl.program_id(1)

    @pl.when(jnp.logical_and(step == 0, ch == 0))
    def _init():
        h_ref[...] = jnp.zeros((BK, HID), jnp.float32)
        c_ref[...] = jnp.zeros((BK, HID), jnp.float32)
        for b in range(B):
            for j in range(K):
                tok_smem[b, j] = SOS_ID

    @pl.when(ch == 0)
    def _pre():
        copies = [
            pltpu.make_async_copy(
                emb_hbm.at[pl.ds(tok_smem[i // K, i % K], 1), :],
                x_ref.at[pl.ds(i, 1), :],
                gsem,
            )
            for i in range(BK)
        ]
        for cp in copies:
            cp.start()
        for cp in copies:
            cp.wait()

        x = x_ref[...]
        h = h_ref[...]
        c = c_ref[...]
        gates = (jnp.dot(x, wx_ref[...], preferred_element_type=jnp.float32)
                 + jnp.dot(h, wh_ref[...], preferred_element_type=jnp.float32)
                 + b_ref[...])
        i_g = gates[:, 0 * HID:1 * HID]
        f_g = gates[:, 1 * HID:2 * HID]
        g_g = gates[:, 2 * HID:3 * HID]
        o_g = gates[:, 3 * HID:4 * HID]
        c = jax.nn.sigmoid(f_g) * c + jax.nn.sigmoid(i_g) * jnp.tanh(g_g)
        h = jax.nn.sigmoid(o_g) * jnp.tanh(c)
        h_ref[...] = h
        c_ref[...] = c

        scs = []
        for src in range(B):
            hb = h[src * K:(src + 1) * K, :]
            eb = encT_ref[:, src * SRC:(src + 1) * SRC]
            scs.append(jnp.dot(hb, eb, preferred_element_type=jnp.float32))
        sc = jnp.concatenate(scs, axis=0)
        mx = jnp.max(sc, axis=1, keepdims=True)
        e = jnp.exp(sc - mx)
        attn = e / jnp.sum(e, axis=1, keepdims=True)
        ctxs = []
        for src in range(B):
            ab = attn[src * K:(src + 1) * K, :]
            eb = enc2_ref[src * SRC:(src + 1) * SRC, :]
            ctxs.append(jnp.dot(ab, eb, preferred_element_type=jnp.float32))
        ctx = jnp.concatenate(ctxs, axis=0)
        cat = jnp.concatenate([h, ctx], axis=1)
        comb_ref[...] = jnp.tanh(
            jnp.dot(cat, wc_ref[...], preferred_element_type=jnp.float32))

        m_ref[...] = jnp.full((BK, 1), NEG, jnp.float32)
        v1_ref[...] = jnp.full((BK, 1), NEG, jnp.float32)
        v2_ref[...] = jnp.full((BK, 1), NEG, jnp.float32)
        i1_ref[...] = jnp.zeros((BK, 1), jnp.int32)
        i2_ref[...] = jnp.zeros((BK, 1), jnp.int32)

    xblk = jnp.dot(comb_ref[...], wo_ref[...],
                   preferred_element_type=jnp.float32)
    colg = ch * CHUNK + jax.lax.broadcasted_iota(jnp.int32, (BK, CHUNK), 1)
    xblk = jnp.where(colg < VOCAB, xblk, NEG)
    logits_ref[:, pl.ds(ch * CHUNK, CHUNK)] = xblk

    cm = jnp.max(xblk, axis=1, keepdims=True)
    m_ref[...] = jnp.maximum(m_ref[...], cm)

    BIGI = jnp.int32(2 ** 30)
    c1i = jnp.min(jnp.where(xblk == cm, colg, BIGI), axis=1, keepdims=True)
    x2 = jnp.where(colg == c1i, NEG, xblk)
    c2v = jnp.max(x2, axis=1, keepdims=True)
    c2i = jnp.min(jnp.where(x2 == c2v, colg, BIGI), axis=1, keepdims=True)
    c1v = cm

    a1 = v1_ref[...]
    a2 = v2_ref[...]
    ai1 = i1_ref[...]
    ai2 = i2_ref[...]
    awin = a1 >= c1v
    sa = a2 >= c1v
    sb = a1 >= c2v
    v1_ref[...] = jnp.where(awin, a1, c1v)
    i1_ref[...] = jnp.where(awin, ai1, c1i)
    v2_ref[...] = jnp.where(awin, jnp.where(sa, a2, c1v),
                            jnp.where(sb, a1, c2v))
    i2_ref[...] = jnp.where(awin, jnp.where(sa, ai2, c1i),
                            jnp.where(sb, ai1, c2i))

    @pl.when(ch == NCHUNK - 1)
    def _merge():
        m_fin = m_ref[...]
        s = jnp.sum(jnp.exp(logits_ref[:, :VPAD] - m_fin),
                    axis=1, keepdims=True)
        lse = jnp.log(s)
        lp1 = (v1_ref[...] - m_fin) - lse
        lp2 = (v2_ref[...] - m_fin) - lse
        t1 = i1_ref[...]
        t2 = i2_ref[...]
        swap = jnp.logical_and(lp1 == lp2, t1 > t2)
        t1n = jnp.where(swap, t2, t1)
        t2n = jnp.where(swap, t1, t2)

        lp1r = lp1.reshape(B, K)
        lp2r = lp2.reshape(B, K)
        t1r = t1n.astype(jnp.float32).reshape(B, K)
        t2r = t2n.astype(jnp.float32).reshape(B, K)
        lp1_0 = lp1r[:, 0:1]
        lp1_1 = lp1r[:, 1:2]
        lp2_0 = lp2r[:, 0:1]
        lp2_1 = lp2r[:, 1:2]
        t1_0 = t1r[:, 0:1]
        t1_1 = t1r[:, 1:2]
        t2_0 = t2r[:, 0:1]
        t2_1 = t2r[:, 1:2]

        pos64 = jax.lax.broadcasted_iota(jnp.int32, (B, 64), 1)

        @pl.when(step == 0)
        def _s0():
            cum_ref[:, 0:1] = lp1_0
            cum_ref[:, 1:2] = lp2_0
            tk0 = t1_0.astype(jnp.int32)
            tk1 = t2_0.astype(jnp.int32)
            tokv_ref[:, 0:1] = tk0
            tokv_ref[:, 1:2] = tk1
            hist_ref[:, 0:64] = jnp.where(pos64 == 0, tk0, 0)
            hist_ref[:, 64:128] = jnp.where(pos64 == 0, tk1, 0)

        @pl.when(step > 0)
        def _sn():
            cum0 = cum_ref[:, 0:1]
            cum1 = cum_ref[:, 1:2]
            cand_v = jnp.concatenate(
                [lp1_0 + cum0, lp2_0 + cum0, lp1_1 + cum1, lp2_1 + cum1],
                axis=1)
            cand_t = jnp.concatenate([t1_0, t2_0, t1_1, t2_1], axis=1)
            i4 = jax.lax.broadcasted_iota(jnp.int32, (B, 4), 1)
            b1 = jnp.max(cand_v, axis=1, keepdims=True)
            s1 = jnp.min(jnp.where(cand_v == b1, i4, 99), axis=1, keepdims=True)
            cv2 = jnp.where(i4 == s1, NEG, cand_v)
            b2 = jnp.max(cv2, axis=1, keepdims=True)
            s2 = jnp.min(jnp.where(cv2 == b2, i4, 99), axis=1, keepdims=True)
            tk0 = jnp.sum(jnp.where(i4 == s1, cand_t, 0.0), axis=1,
                          keepdims=True).astype(jnp.int32)
            tk1 = jnp.sum(jnp.where(i4 == s2, cand_t, 0.0), axis=1,
                          keepdims=True).astype(jnp.int32)
            prev0 = s1 // K
            prev1 = s2 // K

            h0 = hist_ref[:, 0:64]
            h1 = hist_ref[:, 64:128]
            ph0 = jnp.where(prev0 == 0, h0, h1)
            ph1 = jnp.where(prev1 == 0, h0, h1)
            hist_ref[:, 0:64] = jnp.where(pos64 == step, tk0, ph0)
            hist_ref[:, 64:128] = jnp.where(pos64 == step, tk1, ph1)
            cum_ref[:, 0:1] = b1
            cum_ref[:, 1:2] = b2
            tokv_ref[:, 0:1] = tk0
            tokv_ref[:, 1:2] = tk1

        cp = pltpu.make_async_copy(tokv_ref, tok_smem, csem)
        cp.start()
        cp.wait()

        @pl.when(step == STEPS - 1)
        def _out():
            out0_ref[...] = hist_ref[:, 0:MAXLEN]
            out1_ref[...] = hist_ref[:, 64:64 + MAXLEN]


def kernel(input_var, encoder_outputs, k, emb, Wx, Wh, b, Wc, Wo):
    del input_var, k
    enc2 = encoder_outputs.reshape(B * SRC, HID)
    encT = enc2.T
    b2 = b.reshape(1, 4 * HID)

    grid = (STEPS, NCHUNK)
    beams = pl.pallas_call(
        _body,
        grid=grid,
        in_specs=[
            pl.BlockSpec((HID, B * SRC), lambda s, c: (0, 0)),
            pl.BlockSpec((B * SRC, HID), lambda s, c: (0, 0)),
            pl.BlockSpec((HID, 4 * HID), lambda s, c: (0, 0)),
            pl.BlockSpec((HID, 4 * HID), lambda s, c: (0, 0)),
            pl.BlockSpec((1, 4 * HID), lambda s, c: (0, 0)),
            pl.BlockSpec((2 * HID, HID), lambda s, c: (0, 0)),
            pl.BlockSpec((HID, CHUNK), lambda s, c: (0, c)),
            pl.BlockSpec(memory_space=pl.ANY),
        ],
        out_specs=[pl.BlockSpec((B, MAXLEN), lambda s, c: (0, 0)),
                   pl.BlockSpec((B, MAXLEN), lambda s, c: (0, 0))],
        out_shape=[jax.ShapeDtypeStruct((B, MAXLEN), jnp.int32),
                   jax.ShapeDtypeStruct((B, MAXLEN), jnp.int32)],
        scratch_shapes=[
            pltpu.VMEM((BK, HID), jnp.float32),
            pltpu.VMEM((BK, HID), jnp.float32),
            pltpu.VMEM((BK, HID), jnp.float32),
            pltpu.VMEM((BK, HID), jnp.float32),
            pltpu.VMEM((B, K), jnp.float32),
            pltpu.VMEM((BK, VSTORE), jnp.float32),
            pltpu.VMEM((BK, 1), jnp.float32),
            pltpu.VMEM((BK, 1), jnp.float32),
            pltpu.VMEM((BK, 1), jnp.float32),
            pltpu.VMEM((BK, 1), jnp.int32),
            pltpu.VMEM((BK, 1), jnp.int32),
            pltpu.VMEM((B, 128), jnp.int32),
            pltpu.VMEM((B, K), jnp.int32),
            pltpu.SMEM((B, K), jnp.int32),
            pltpu.SemaphoreType.DMA,
            pltpu.SemaphoreType.DMA,
        ],
        compiler_params=pltpu.CompilerParams(
            dimension_semantics=("arbitrary", "arbitrary"),
        ),
    )(encT, enc2, Wx, Wh, b2, Wc, Wo, emb)
    return jnp.stack(beams, axis=1)
